# initial kernel scaffold (unmeasured)
import jax
import jax.numpy as jnp
from jax import lax
from jax.experimental import pallas as pl
from jax.experimental.pallas import tpu as pltpu

N_DEV = 32


def kernel(Q, K, V):
    b, s, h, d = Q.shape
    B = b * h
    scale = d ** -0.5

    Qr = jnp.transpose(Q, (0, 2, 1, 3)).reshape(B, s, d).astype(jnp.bfloat16)
    Kr = jnp.transpose(K, (0, 2, 1, 3)).reshape(B, s, d).astype(jnp.bfloat16)
    Vr = jnp.transpose(V, (0, 2, 1, 3)).reshape(B, s, d).astype(jnp.bfloat16)

    def body(q_ref, k_ref, v_ref, out_ref, kcomm, vcomm,
             send_sems, recv_sems, credit_sem):
        my = lax.axis_index("i")
        left = lax.rem(my + N_DEV - 1, N_DEV)
        right = lax.rem(my + 1, N_DEV)

        barrier = pltpu.get_barrier_semaphore()
        for nbr in (left, right):
            pl.semaphore_signal(
                barrier, inc=1,
                device_id=(nbr,), device_id_type=pl.DeviceIdType.MESH,
            )
        pl.semaphore_wait(barrier, 2)

        kcomm[0, ...] = k_ref[...]
        vcomm[0, ...] = v_ref[...]

        q = q_ref[...]

        def process(k_c, v_c, m, l, acc):
            sc = lax.dot_general(
                q, k_c, (((2,), (2,)), ((0,), (0,))),
                preferred_element_type=jnp.float32,
            ) * scale
            m_new = jnp.maximum(m, sc.max(-1))
            p = jnp.exp(sc - m_new[:, :, None])
            alpha = jnp.exp(m - m_new)
            l_new = l * alpha + p.sum(-1)
            pv = lax.dot_general(
                p.astype(jnp.bfloat16), v_c, (((2,), (1,)), ((0,), (0,))),
                preferred_element_type=jnp.float32,
            )
            return m_new, l_new, acc * alpha[:, :, None] + pv

        m0 = jnp.full((B, s), -1e30, jnp.float32)
        l0 = jnp.zeros((B, s), jnp.float32)
        a0 = jnp.zeros((B, s, d), jnp.float32)
        m, l, acc = process(k_ref[...], v_ref[...], m0, l0, a0)

        def hop(hi, carry):
            m, l, acc = carry
            cur = lax.rem(hi, 2)
            nxt = lax.rem(hi + 1, 2)

            @pl.when(hi >= 1)
            def _():
                pl.semaphore_wait(credit_sem, 1)

            rk = pltpu.make_async_remote_copy(
                src_ref=kcomm.at[cur], dst_ref=kcomm.at[nxt],
                send_sem=send_sems.at[0, cur], recv_sem=recv_sems.at[0, nxt],
                device_id=(right,), device_id_type=pl.DeviceIdType.MESH,
            )
            rv = pltpu.make_async_remote_copy(
                src_ref=vcomm.at[cur], dst_ref=vcomm.at[nxt],
                send_sem=send_sems.at[1, cur], recv_sem=recv_sems.at[1, nxt],
                device_id=(right,), device_id_type=pl.DeviceIdType.MESH,
            )
            rk.start()
            rv.start()
            rk.wait()
            rv.wait()

            @pl.when(hi <= N_DEV - 3)
            def _():
                pl.semaphore_signal(
                    credit_sem, inc=1,
                    device_id=(left,), device_id_type=pl.DeviceIdType.MESH,
                )

            return process(kcomm[nxt], vcomm[nxt], m, l, acc)

        m, l, acc = lax.fori_loop(0, N_DEV - 1, hop, (m, l, acc))
        out_ref[...] = acc / l[:, :, None]

    out = pl.pallas_call(
        body,
        out_shape=jax.ShapeDtypeStruct((B, s, d), jnp.float32),
        in_specs=[pl.BlockSpec(memory_space=pltpu.VMEM)] * 3,
        out_specs=pl.BlockSpec(memory_space=pltpu.VMEM),
        scratch_shapes=[
            pltpu.VMEM((2, B, s, d), jnp.bfloat16),
            pltpu.VMEM((2, B, s, d), jnp.bfloat16),
            pltpu.SemaphoreType.DMA((2, 2)),
            pltpu.SemaphoreType.DMA((2, 2)),
            pltpu.SemaphoreType.REGULAR,
        ],
        compiler_params=pltpu.CompilerParams(collective_id=0),
    )(Qr, Kr, Vr)

    return out.reshape(b, h, s, d).transpose(0, 2, 1, 3)


# baseline (device time: 1789129 ns/iter reference)
import jax
import jax.numpy as jnp
from jax import lax
from jax.experimental import pallas as pl
from jax.experimental.pallas import tpu as pltpu

N_DEV = 32
BBLK = 8


def kernel(Q, K, V):
    b, s, h, d = Q.shape
    B = b * h
    scale = d ** -0.5

    Qr = jnp.transpose(Q, (0, 2, 1, 3)).reshape(B, s, d).astype(jnp.bfloat16)
    Kt = jnp.transpose(K, (0, 2, 3, 1)).reshape(B, d, s).astype(jnp.bfloat16)
    Vt = jnp.transpose(V, (0, 2, 3, 1)).reshape(B, d, s).astype(jnp.bfloat16)

    def body(q_ref, k_ref, v_ref, out_ref, kcomm, vcomm,
             acc_ref, m_ref, l_ref, send_sems, recv_sems, credit_sem):
        my = lax.axis_index("i")
        left = lax.rem(my + N_DEV - 1, N_DEV)
        right = lax.rem(my + 1, N_DEV)

        barrier = pltpu.get_barrier_semaphore()
        for nbr in (left, right):
            pl.semaphore_signal(
                barrier, inc=1,
                device_id=(nbr,), device_id_type=pl.DeviceIdType.MESH,
            )
        pl.semaphore_wait(barrier, 2)

        kcomm[0, ...] = k_ref[...]
        vcomm[0, ...] = v_ref[...]

        m_ref[...] = jnp.full((B, s), -1e30, jnp.float32)
        l_ref[...] = jnp.zeros((B, s), jnp.float32)
        acc_ref[...] = jnp.zeros((B, d, s), jnp.float32)

        def process_slot(slot):
            def blk(i, _):
                bs = pl.ds(i * BBLK, BBLK)
                qb = q_ref[bs]
                kb = kcomm[slot, bs]
                vb = vcomm[slot, bs]
                sc = lax.dot_general(
                    qb, kb, (((2,), (1,)), ((0,), (0,))),
                    preferred_element_type=jnp.float32,
                ) * scale
                m_prev = m_ref[bs]
                m_new = jnp.maximum(m_prev, sc.max(-1))
                p = jnp.exp(sc - m_new[:, :, None])
                alpha = jnp.exp(m_prev - m_new)
                l_ref[bs] = l_ref[bs] * alpha + p.sum(-1)
                pv = lax.dot_general(
                    vb, p.astype(jnp.bfloat16), (((2,), (2,)), ((0,), (0,))),
                    preferred_element_type=jnp.float32,
                )
                acc_ref[bs] = acc_ref[bs] * alpha[:, None, :] + pv
                m_ref[bs] = m_new
                return 0

            lax.fori_loop(0, B // BBLK, blk, 0)

        process_slot(0)

        def hop(hi, _):
            cur = lax.rem(hi, 2)
            nxt = lax.rem(hi + 1, 2)

            @pl.when(hi >= 1)
            def _():
                pl.semaphore_wait(credit_sem, 1)

            rk = pltpu.make_async_remote_copy(
                src_ref=kcomm.at[cur], dst_ref=kcomm.at[nxt],
                send_sem=send_sems.at[0, cur], recv_sem=recv_sems.at[0, nxt],
                device_id=(right,), device_id_type=pl.DeviceIdType.MESH,
            )
            rv = pltpu.make_async_remote_copy(
                src_ref=vcomm.at[cur], dst_ref=vcomm.at[nxt],
                send_sem=send_sems.at[1, cur], recv_sem=recv_sems.at[1, nxt],
                device_id=(right,), device_id_type=pl.DeviceIdType.MESH,
            )
            rk.start()
            rv.start()
            rk.wait()
            rv.wait()

            @pl.when(hi <= N_DEV - 3)
            def _():
                pl.semaphore_signal(
                    credit_sem, inc=1,
                    device_id=(left,), device_id_type=pl.DeviceIdType.MESH,
                )

            process_slot(nxt)
            return 0

        lax.fori_loop(0, N_DEV - 1, hop, 0)

        out_ref[...] = acc_ref[...] / l_ref[...][:, None, :]

    out = pl.pallas_call(
        body,
        out_shape=jax.ShapeDtypeStruct((B, d, s), jnp.float32),
        in_specs=[pl.BlockSpec(memory_space=pltpu.VMEM)] * 3,
        out_specs=pl.BlockSpec(memory_space=pltpu.VMEM),
        scratch_shapes=[
            pltpu.VMEM((2, B, d, s), jnp.bfloat16),
            pltpu.VMEM((2, B, d, s), jnp.bfloat16),
            pltpu.VMEM((B, d, s), jnp.float32),
            pltpu.VMEM((B, s), jnp.float32),
            pltpu.VMEM((B, s), jnp.float32),
            pltpu.SemaphoreType.DMA((2, 2)),
            pltpu.SemaphoreType.DMA((2, 2)),
            pltpu.SemaphoreType.REGULAR,
        ],
        compiler_params=pltpu.CompilerParams(collective_id=0),
    )(Qr, Kt, Vt)

    return out.reshape(b, h, d, s).transpose(0, 3, 1, 2)
